# baseline (speedup 1.0000x reference)
import jax
import jax.numpy as jnp
from jax.experimental import pallas as pl
from jax.experimental.pallas import tpu as pltpu

B, M, H, D, K = 8, 512, 8, 32, 16
DD = D * D
MB = 64
NB = M // MB
RB = MB * H


def _body(m_s, h_s, k_s, starts, ends, probs,
          mat_in, norm_in, mu_ref, nu_ref, main_rep, aux_rep, main_nrm,
          mat_out, norm_out):
    b = pl.program_id(0)
    nb = pl.program_id(1)

    mat_out[...] = mat_in[...]
    norm_out[...] = norm_in[...]

    def upd(i, carry):
        m = m_s[b, i]
        h = h_s[b, i]
        k = k_s[b, i]
        r = (m - nb * MB) * H + h
        p = probs[b, h, k]

        mrep = main_rep[pl.ds(m * H + h, 1), :]
        arep = aux_rep[pl.ds(m, 1), :]
        mat_dec = jax.nn.sigmoid(mrep + arep)

        sel_m = mat_out[pl.ds(r, 1), :]
        mu = mu_ref[pl.ds((b * K + k) * H + h, 1), :]
        mat_out[pl.ds(r, 1), :] = sel_m + (p * mat_dec) * (mu - sel_m)

        mnrm = main_nrm[pl.ds(m * H + h, 1), :]
        norm_dec = jax.nn.sigmoid(mnrm)
        sel_n = norm_out[pl.ds(r, 1), :]
        nu = nu_ref[pl.ds((b * K + k) * H + h, 1), :]
        norm_out[pl.ds(r, 1), :] = sel_n + (p * norm_dec) * (nu - sel_n)
        return carry

    jax.lax.fori_loop(starts[b, nb], ends[b, nb], upd, 0)


def kernel(matrix, normalizer, matrix_update, normalizer_update,
           main_decay_logits, aux_decay_logits, sel_index, sel_probs):
    mat2 = matrix.reshape(B * M * H, DD)
    norm2 = normalizer.reshape(B * M * H, D)
    mu2 = matrix_update.reshape(B * K * H, DD)
    nu2 = normalizer_update.reshape(B * K * H, D)
    main_rep = jnp.repeat(main_decay_logits.reshape(M * H, D), D, axis=1)
    aux_rep = jnp.tile(aux_decay_logits.reshape(M, D), (1, D))
    main_nrm = main_decay_logits.reshape(M * H, D)

    m_all = sel_index.reshape(B, H * K)
    order = jnp.argsort(m_all, axis=1).astype(jnp.int32)
    m_sorted = jnp.take_along_axis(m_all, order, axis=1).astype(jnp.int32)
    h_sorted = order // K
    k_sorted = order % K
    bounds = jnp.arange(NB + 1, dtype=jnp.int32) * MB
    pos = jax.vmap(lambda row: jnp.searchsorted(row, bounds, side='left'))(
        m_sorted).astype(jnp.int32)
    starts, ends = pos[:, :-1], pos[:, 1:]

    def mem_map(b, nb, *_):
        return (b * NB + nb, 0)

    def whole(*_):
        return (0, 0)

    grid_spec = pltpu.PrefetchScalarGridSpec(
        num_scalar_prefetch=6,
        grid=(B, NB),
        in_specs=[
            pl.BlockSpec((RB, DD), mem_map),
            pl.BlockSpec((RB, D), mem_map),
            pl.BlockSpec((B * K * H, DD), whole),
            pl.BlockSpec((B * K * H, D), whole),
            pl.BlockSpec((M * H, DD), whole),
            pl.BlockSpec((M, DD), whole),
            pl.BlockSpec((M * H, D), whole),
        ],
        out_specs=[
            pl.BlockSpec((RB, DD), mem_map),
            pl.BlockSpec((RB, D), mem_map),
        ],
    )

    out_mat, out_norm = pl.pallas_call(
        _body,
        grid_spec=grid_spec,
        out_shape=[
            jax.ShapeDtypeStruct(mat2.shape, mat2.dtype),
            jax.ShapeDtypeStruct(norm2.shape, norm2.dtype),
        ],
    )(m_sorted, h_sorted, k_sorted, starts, ends, sel_probs,
      mat2, norm2, mu2, nu2, main_rep, aux_rep, main_nrm)

    return (out_mat.reshape(B, M, H, D, D), out_norm.reshape(B, M, H, D))


# speedup vs baseline: 1.4555x; 1.4555x over previous
import jax
import jax.numpy as jnp
from jax.experimental import pallas as pl
from jax.experimental.pallas import tpu as pltpu

B, M, H, D, K = 8, 512, 8, 32, 16
MB = 32
NB = M // MB


def _body(m_s, h_s, k_s, starts, ends, probs,
          mat_in, norm_in, mu_ref, nu_ref, main_ref, aux_ref,
          mat_out, norm_out):
    b = pl.program_id(0)
    nb = pl.program_id(1)

    mat_out[...] = mat_in[...]
    norm_out[...] = norm_in[...]

    def upd(i, carry):
        m = m_s[b, i]
        h = h_s[b, i]
        k = k_s[b, i]
        m_rel = m - nb * MB
        p = probs[b, h, k]

        mrow = main_ref[pl.ds(m, 1), h]
        mcol = jnp.swapaxes(mrow, 0, 1)
        arow = aux_ref[pl.ds(m, 1)]
        mat_dec = jax.nn.sigmoid(mcol + arow)
        norm_dec = jax.nn.sigmoid(mrow)

        sel_m = mat_out[0, m_rel, h]
        mu = mu_ref[0, k, h]
        mat_out[0, m_rel, h] = sel_m + (p * mat_dec) * (mu - sel_m)

        sel_n = norm_out[0, pl.ds(m_rel, 1), h]
        nu = nu_ref[0, pl.ds(k, 1), h]
        norm_out[0, pl.ds(m_rel, 1), h] = sel_n + (p * norm_dec) * (nu - sel_n)
        return carry

    jax.lax.fori_loop(starts[b, nb], ends[b, nb], upd, 0)


def kernel(matrix, normalizer, matrix_update, normalizer_update,
           main_decay_logits, aux_decay_logits, sel_index, sel_probs):
    aux2 = aux_decay_logits.reshape(M, D)

    m_all = sel_index.reshape(B, H * K)
    order = jnp.argsort(m_all, axis=1).astype(jnp.int32)
    m_sorted = jnp.take_along_axis(m_all, order, axis=1).astype(jnp.int32)
    h_sorted = order // K
    k_sorted = order % K
    bounds = jnp.arange(NB + 1, dtype=jnp.int32) * MB
    pos = jax.vmap(lambda row: jnp.searchsorted(row, bounds, side='left'))(
        m_sorted).astype(jnp.int32)
    starts, ends = pos[:, :-1], pos[:, 1:]

    def mem_map(b, nb, *_):
        return (b, nb, 0, 0, 0)

    def nrm_map(b, nb, *_):
        return (b, nb, 0, 0)

    def upd_map(b, nb, *_):
        return (b, 0, 0, 0, 0)

    def upd_nrm_map(b, nb, *_):
        return (b, 0, 0, 0)

    def whole3(*_):
        return (0, 0, 0)

    def whole2(*_):
        return (0, 0)

    grid_spec = pltpu.PrefetchScalarGridSpec(
        num_scalar_prefetch=6,
        grid=(B, NB),
        in_specs=[
            pl.BlockSpec((1, MB, H, D, D), mem_map),
            pl.BlockSpec((1, MB, H, D), nrm_map),
            pl.BlockSpec((1, K, H, D, D), upd_map),
            pl.BlockSpec((1, K, H, D), upd_nrm_map),
            pl.BlockSpec((M, H, D), whole3),
            pl.BlockSpec((M, D), whole2),
        ],
        out_specs=[
            pl.BlockSpec((1, MB, H, D, D), mem_map),
            pl.BlockSpec((1, MB, H, D), nrm_map),
        ],
    )

    out_mat, out_norm = pl.pallas_call(
        _body,
        grid_spec=grid_spec,
        out_shape=[
            jax.ShapeDtypeStruct(matrix.shape, matrix.dtype),
            jax.ShapeDtypeStruct(normalizer.shape, normalizer.dtype),
        ],
    )(m_sorted, h_sorted, k_sorted, starts, ends, sel_probs,
      matrix, normalizer, matrix_update, normalizer_update,
      main_decay_logits, aux2)

    return (out_mat, out_norm)


# speedup vs baseline: 1.4867x; 1.0214x over previous
import jax
import jax.numpy as jnp
from jax.experimental import pallas as pl
from jax.experimental.pallas import tpu as pltpu

B, M, H, D, K = 8, 512, 8, 32, 16
MB = 64
NB = M // MB


def _body(m_s, h_s, k_s, starts, ends, probs,
          mat_in, norm_in, mu_ref, nu_ref, main_ref, aux_ref,
          mat_out, norm_out):
    b = pl.program_id(0)
    nb = pl.program_id(1)

    mat_out[...] = mat_in[...]
    norm_out[...] = norm_in[...]

    def upd(i, carry):
        m = m_s[b, i]
        h = h_s[b, i]
        k = k_s[b, i]
        m_rel = m - nb * MB
        p = probs[b, h, k]

        mrow = main_ref[pl.ds(m, 1), h]
        mcol = jnp.swapaxes(mrow, 0, 1)
        arow = aux_ref[pl.ds(m, 1)]
        mat_dec = jax.nn.sigmoid(mcol + arow)
        norm_dec = jax.nn.sigmoid(mrow)

        sel_m = mat_out[0, m_rel, h]
        mu = mu_ref[0, k, h]
        mat_out[0, m_rel, h] = sel_m + (p * mat_dec) * (mu - sel_m)

        sel_n = norm_out[0, pl.ds(m_rel, 1), h]
        nu = nu_ref[0, pl.ds(k, 1), h]
        norm_out[0, pl.ds(m_rel, 1), h] = sel_n + (p * norm_dec) * (nu - sel_n)
        return carry

    jax.lax.fori_loop(starts[b, nb], ends[b, nb], upd, 0)


def kernel(matrix, normalizer, matrix_update, normalizer_update,
           main_decay_logits, aux_decay_logits, sel_index, sel_probs):
    aux2 = aux_decay_logits.reshape(M, D)

    m_all = sel_index.reshape(B, H * K)
    order = jnp.argsort(m_all, axis=1).astype(jnp.int32)
    m_sorted = jnp.take_along_axis(m_all, order, axis=1).astype(jnp.int32)
    h_sorted = order // K
    k_sorted = order % K
    bounds = jnp.arange(NB + 1, dtype=jnp.int32) * MB
    pos = jax.vmap(lambda row: jnp.searchsorted(row, bounds, side='left'))(
        m_sorted).astype(jnp.int32)
    starts, ends = pos[:, :-1], pos[:, 1:]

    def mem_map(b, nb, *_):
        return (b, nb, 0, 0, 0)

    def nrm_map(b, nb, *_):
        return (b, nb, 0, 0)

    def upd_map(b, nb, *_):
        return (b, 0, 0, 0, 0)

    def upd_nrm_map(b, nb, *_):
        return (b, 0, 0, 0)

    def whole3(*_):
        return (0, 0, 0)

    def whole2(*_):
        return (0, 0)

    grid_spec = pltpu.PrefetchScalarGridSpec(
        num_scalar_prefetch=6,
        grid=(B, NB),
        in_specs=[
            pl.BlockSpec((1, MB, H, D, D), mem_map),
            pl.BlockSpec((1, MB, H, D), nrm_map),
            pl.BlockSpec((1, K, H, D, D), upd_map),
            pl.BlockSpec((1, K, H, D), upd_nrm_map),
            pl.BlockSpec((M, H, D), whole3),
            pl.BlockSpec((M, D), whole2),
        ],
        out_specs=[
            pl.BlockSpec((1, MB, H, D, D), mem_map),
            pl.BlockSpec((1, MB, H, D), nrm_map),
        ],
    )

    out_mat, out_norm = pl.pallas_call(
        _body,
        grid_spec=grid_spec,
        out_shape=[
            jax.ShapeDtypeStruct(matrix.shape, matrix.dtype),
            jax.ShapeDtypeStruct(normalizer.shape, normalizer.dtype),
        ],
    )(m_sorted, h_sorted, k_sorted, starts, ends, sel_probs,
      matrix, normalizer, matrix_update, normalizer_update,
      main_decay_logits, aux2)

    return (out_mat, out_norm)
